# baseline (speedup 1.0000x reference)
import functools

import jax
import jax.numpy as jnp
from jax import lax
from jax.experimental import pallas as pl
from jax.experimental.pallas import tpu as pltpu
from jax.experimental.pallas import tpu_sc as plsc

NC, NS, L = 2, 16, 16
NW = NC * NS
CH = 128
NEG = -1e30
H1, C1 = 8, 8


def _perm(v, idx):
    dn = lax.GatherDimensionNumbers(
        offset_dims=(), collapsed_slice_dims=(0,), start_index_map=(0,))
    return lax.gather(v, idx[:, None], dn, slice_sizes=(1,),
                      mode=lax.GatherScatterMode.PROMISE_IN_BOUNDS)


def _splat(x):
    return jnp.full((L,), x, jnp.int32)


def _tc0_body(x_ref, w_ref, asf_ref, adf_ref, h_ref, es_ref, ed_ref, g_ref,
              *, n, npad):
    x = x_ref[...]
    h = jnp.dot(x, w_ref[...], preferred_element_type=jnp.float32)
    h_ref[...] = jnp.concatenate(
        [h, jnp.zeros((npad, 128 - H1 * C1), jnp.float32)], axis=1)
    kk = lax.broadcasted_iota(jnp.int32, (H1 * C1, H1), 0)
    hh = lax.broadcasted_iota(jnp.int32, (H1 * C1, H1), 1)
    blk = (kk // C1) == hh
    A_s = jnp.where(blk, jnp.broadcast_to(asf_ref[...].reshape(H1 * C1, 1),
                                          (H1 * C1, H1)), 0.0)
    A_d = jnp.where(blk, jnp.broadcast_to(adf_ref[...].reshape(H1 * C1, 1),
                                          (H1 * C1, H1)), 0.0)
    es = jnp.dot(h, A_s, preferred_element_type=jnp.float32)
    ed = jnp.dot(h, A_d, preferred_element_type=jnp.float32)
    valid = lax.broadcasted_iota(jnp.int32, (npad, H1), 0) < n
    esm = jnp.where(valid, es, NEG)
    edm = jnp.where(valid, ed, NEG)
    g = jnp.maximum(0.0, jnp.max(esm, axis=0) + jnp.max(edm, axis=0))
    g_ref[...] = jnp.concatenate([g, g])[None, :]
    es_ref[...] = esm
    ed_ref[...] = edm


def _merge1_body(op_ref, b1_ref, w2_ref, as2_ref, ad2_ref,
                 h2_ref, es2_ref, ed2_ref, g2_ref, *, n, npad):
    P = op_ref[0:npad] + op_ref[npad:2 * npad]
    acc = P[:, 0:H1 * C1]
    kk = lax.broadcasted_iota(jnp.int32, (128, H1), 0)
    hh = lax.broadcasted_iota(jnp.int32, (128, H1), 1)
    sel = kk == (64 + 8 * hh)
    Ssel = jnp.where(sel, 1.0, 0.0)
    s8 = jnp.dot(P, Ssel, preferred_element_type=jnp.float32)
    r8 = 1.0 / (s8 + 1e-16)
    hh2 = lax.broadcasted_iota(jnp.int32, (H1, H1 * C1), 0)
    kk2 = lax.broadcasted_iota(jnp.int32, (H1, H1 * C1), 1)
    Em = jnp.where(hh2 == (kk2 // C1), 1.0, 0.0)
    R = jnp.dot(r8, Em, preferred_element_type=jnp.float32)
    h2in = jnp.maximum(acc * R + b1_ref[...], 0.0)
    h2 = jnp.dot(h2in, w2_ref[...], preferred_element_type=jnp.float32)
    h2_ref[...] = h2
    es2 = jnp.dot(h2, as2_ref[...], preferred_element_type=jnp.float32)
    ed2 = jnp.dot(h2, ad2_ref[...], preferred_element_type=jnp.float32)
    valid = lax.broadcasted_iota(jnp.int32, (npad, 1), 0) < n
    es2m = jnp.where(valid, es2, NEG)
    ed2m = jnp.where(valid, ed2, NEG)
    g2 = jnp.maximum(0.0, jnp.max(es2m) + jnp.max(ed2m))
    g2_ref[...] = jnp.zeros((1, 16), jnp.float32) + g2
    es2_ref[...] = es2m
    ed2_ref[...] = ed2m


def _final_body(sp_ref, op_ref, b2_ref, o_ref, *, npad):
    s = sp_ref[0:npad, 0:1] + sp_ref[npad:2 * npad, 0:1]
    r = 1.0 / (s + 1e-16)
    acc = op_ref[0:npad] + op_ref[npad:2 * npad]
    out = acc * r + b2_ref[...]
    z = out - jnp.max(out, axis=1, keepdims=True)
    o_ref[...] = z - jnp.log(jnp.sum(jnp.exp(z), axis=1, keepdims=True))


def _sc1_body(srcp, dstp, tab_hbm, g, exlo_out, exhi_out,
              src_vm, dst_vm, tab_vm, exbuf, g_vm, *, nch, npad):
    cid = lax.axis_index("c")
    sid = lax.axis_index("s")
    wid = sid * NC + cid
    pltpu.sync_copy(srcp.at[wid], src_vm)
    pltpu.sync_copy(dstp.at[wid], dst_vm)
    pltpu.sync_copy(g, g_vm)
    gv = g_vm[...]
    wpn = 2 * 4
    for p, ex_out in ((0, exlo_out), (1, exhi_out)):
        pltpu.sync_copy(tab_hbm.at[pl.ds(p * npad * wpn, npad * wpn)], tab_vm)
        gj = [_perm(gv, _splat(4 * p + j)) for j in range(4)]

        def chunk(ch, carry):
            def group(gi, c2):
                src16 = src_vm[ch, pl.ds(gi * L, L)]
                dst16 = dst_vm[ch, pl.ds(gi * L, L)]
                for j in range(4):
                    es16 = plsc.load_gather(tab_vm, [src16 * wpn + j])
                    ed16 = plsc.load_gather(tab_vm, [dst16 * wpn + 4 + j])
                    t = es16 + ed16
                    t = jnp.maximum(t, 0.2 * t)
                    exbuf[pl.ds(j * CH + gi * L, L)] = jnp.exp(t - gj[j])
                return c2

            lax.fori_loop(0, CH // L, group, 0)
            pltpu.sync_copy(
                exbuf, ex_out.at[pl.ds((wid * nch + ch) * 4 * CH, 4 * CH)])
            return carry

        lax.fori_loop(0, nch, chunk, 0)


def _sc2_body(srcp2, dstp2, h_hbm, exlo, exhi, zD, out_parts,
              src_c0, dst_c0, src_c1, dst_c1, rowb0, rowb1,
              exl0, exh0, exl1, exh1, out_acc, sem0, sem1, semS0, semS1,
              *, nch, rpt, npad):
    cid = lax.axis_index("c")
    sid = lax.axis_index("s")
    wid = sid * NC + cid
    pltpu.sync_copy(zD.at[pl.ds(sid * rpt, rpt)],
                    out_acc.at[pl.ds(sid * rpt, rpt)])
    plsc.subcore_barrier()
    himask = lax.iota(jnp.int32, L) >= 8

    def fire(ch, src_c, dst_c, rowb, exl, exh, sem):
        pltpu.sync_copy(srcp2.at[wid * nch + ch], src_c)
        pltpu.sync_copy(dstp2.at[wid * nch + ch], dst_c)
        pltpu.async_copy(h_hbm.at[src_c], rowb, sem)
        pltpu.async_copy(exlo.at[pl.ds((wid * nch + ch) * 4 * CH, 4 * CH)],
                         exl, sem)
        pltpu.async_copy(exhi.at[pl.ds((wid * nch + ch) * 4 * CH, 4 * CH)],
                         exh, sem)

    def drain(src_c, rowb, exl, exh, sem):
        pltpu.make_async_copy(h_hbm.at[src_c], rowb, sem).wait()
        pltpu.make_async_copy(exlo.at[pl.ds(0, 4 * CH)], exl, sem).wait()
        pltpu.make_async_copy(exhi.at[pl.ds(0, 4 * CH)], exh, sem).wait()

    def compute(rowb, exl, exh, dst_c, semS):
        def group(gi, c2):
            eh = [exl[pl.ds(h * CH + gi * L, L)] for h in range(4)] + \
                 [exh[pl.ds(h * CH + gi * L, L)] for h in range(4)]
            for k in range(L):
                e = gi * L + k
                for j in range(4):
                    a = jnp.where(himask,
                                  _perm(eh[2 * j + 1], _splat(k)),
                                  _perm(eh[2 * j], _splat(k)))
                    rowb[e, pl.ds(j * L, L)] = rowb[e, pl.ds(j * L, L)] * a
                    rowb[e, pl.ds(64 + j * L, L)] = a
            return c2

        lax.fori_loop(0, CH // L, group, 0)
        pltpu.async_copy(rowb, out_acc.at[dst_c], semS, add=True)

    set0 = (src_c0, dst_c0, rowb0, exl0, exh0, sem0, semS0)
    set1 = (src_c1, dst_c1, rowb1, exl1, exh1, sem1, semS1)

    def drain_sc(s):
        src_c, dst_c, rowb, exl, exh, sem, semS = s
        pltpu.make_async_copy(rowb, out_acc.at[dst_c], semS).wait()

    def fire_s(ch, s, first):
        src_c, dst_c, rowb, exl, exh, sem, semS = s
        if not first:
            drain_sc(s)
        fire(ch, src_c, dst_c, rowb, exl, exh, sem)

    def use(s):
        src_c, dst_c, rowb, exl, exh, sem, semS = s
        drain(src_c, rowb, exl, exh, sem)
        compute(rowb, exl, exh, dst_c, semS)

    if nch % 2 == 1 and nch > 2:
        fire_s(0, set0, True)
        fire_s(1, set1, True)
        use(set0)
        fire_s(2, set0, False)
        use(set1)

        def pair(i, carry):
            fire_s(2 * i + 1, set1, False)
            use(set0)
            fire_s(2 * i + 2, set0, False)
            use(set1)
            return carry

        lax.fori_loop(1, (nch - 1) // 2, pair, 0)
        use(set0)
        drain_sc(set0)
        drain_sc(set1)
    else:
        def chunk(ch, carry):
            fire(ch, src_c0, dst_c0, rowb0, exl0, exh0, sem0)
            drain(src_c0, rowb0, exl0, exh0, sem0)

            def group(gi, c2):
                eh = [exl0[pl.ds(h * CH + gi * L, L)] for h in range(4)] + \
                     [exh0[pl.ds(h * CH + gi * L, L)] for h in range(4)]
                for k in range(L):
                    e = gi * L + k
                    for j in range(4):
                        a = jnp.where(himask,
                                      _perm(eh[2 * j + 1], _splat(k)),
                                      _perm(eh[2 * j], _splat(k)))
                        rowb0[e, pl.ds(j * L, L)] = \
                            rowb0[e, pl.ds(j * L, L)] * a
                        rowb0[e, pl.ds(64 + j * L, L)] = a
                return c2

            lax.fori_loop(0, CH // L, group, 0)
            pltpu.sync_copy(rowb0, out_acc.at[dst_c0], add=True)
            return carry

        lax.fori_loop(0, nch, chunk, 0)
    plsc.subcore_barrier()
    pltpu.sync_copy(out_acc.at[pl.ds(sid * rpt, rpt)],
                    out_parts.at[pl.ds(cid * npad + sid * rpt, rpt)])


def _sc3_body(srcp2, dstp2, tab_hbm, g, zD, ex_out, s_parts,
              src_c, dst_c, tab_vm, exbuf, exrowc, g_vm, s_acc,
              *, nch, rpt, npad):
    cid = lax.axis_index("c")
    sid = lax.axis_index("s")
    wid = sid * NC + cid
    pltpu.sync_copy(tab_hbm, tab_vm)
    pltpu.sync_copy(g, g_vm)
    pltpu.sync_copy(zD.at[pl.ds(sid * rpt, rpt)],
                    s_acc.at[pl.ds(sid * rpt, rpt)])

    def zrow(e, c2):
        for j in range(8):
            exrowc[e, pl.ds(j * L, L)] = jnp.zeros((L,), jnp.float32)
        return c2

    lax.fori_loop(0, CH, zrow, 0)
    plsc.subcore_barrier()
    gv = g_vm[...]

    def chunk(ch, carry):
        pltpu.sync_copy(srcp2.at[wid * nch + ch], src_c)
        pltpu.sync_copy(dstp2.at[wid * nch + ch], dst_c)

        def group(gi, c2):
            src16 = src_c[pl.ds(gi * L, L)]
            dst16 = dst_c[pl.ds(gi * L, L)]
            es16 = plsc.load_gather(tab_vm, [src16 * 2])
            ed16 = plsc.load_gather(tab_vm, [dst16 * 2 + 1])
            t = es16 + ed16
            t = jnp.maximum(t, 0.2 * t)
            ex16 = jnp.exp(t - gv)
            exbuf[pl.ds(gi * L, L)] = ex16
            for k in range(L):
                exrowc[gi * L + k, pl.ds(0, L)] = _perm(ex16, _splat(k))
            return c2

        lax.fori_loop(0, CH // L, group, 0)
        pltpu.sync_copy(exbuf, ex_out.at[pl.ds((wid * nch + ch) * CH, CH)])
        pltpu.sync_copy(exrowc, s_acc.at[dst_c], add=True)
        return carry

    lax.fori_loop(0, nch, chunk, 0)
    plsc.subcore_barrier()
    pltpu.sync_copy(s_acc.at[pl.ds(sid * rpt, rpt)],
                    s_parts.at[pl.ds(cid * npad + sid * rpt, rpt)])


def _sc4_body(srcp2, dstp2, h_hbm, ex, zD, out_parts,
              src_c0, dst_c0, src_c1, dst_c1, rowb0, rowb1,
              exb0, exb1, out_acc, sem0, sem1, semS0, semS1,
              *, nch, rpt, npad):
    cid = lax.axis_index("c")
    sid = lax.axis_index("s")
    wid = sid * NC + cid
    pltpu.sync_copy(zD.at[pl.ds(sid * rpt, rpt)],
                    out_acc.at[pl.ds(sid * rpt, rpt)])
    plsc.subcore_barrier()

    set0 = (src_c0, dst_c0, rowb0, exb0, sem0, semS0)
    set1 = (src_c1, dst_c1, rowb1, exb1, sem1, semS1)

    def drain_sc(s):
        src_c, dst_c, rowb, exb, sem, semS = s
        pltpu.make_async_copy(rowb, out_acc.at[dst_c], semS).wait()

    def fire_s(ch, s, first):
        src_c, dst_c, rowb, exb, sem, semS = s
        if not first:
            drain_sc(s)
        pltpu.sync_copy(srcp2.at[wid * nch + ch], src_c)
        pltpu.sync_copy(dstp2.at[wid * nch + ch], dst_c)
        pltpu.async_copy(h_hbm.at[src_c], rowb, sem)
        pltpu.async_copy(ex.at[pl.ds((wid * nch + ch) * CH, CH)], exb, sem)

    def compute(s, sync_scatter=False):
        src_c, dst_c, rowb, exb, sem, semS = s

        def group(gi, c2):
            exg = exb[pl.ds(gi * L, L)]
            for k in range(L):
                e = gi * L + k
                a = _perm(exg, _splat(k))
                for j in range(8):
                    rowb[e, pl.ds(j * L, L)] = rowb[e, pl.ds(j * L, L)] * a
            return c2

        lax.fori_loop(0, CH // L, group, 0)
        if sync_scatter:
            pltpu.sync_copy(rowb, out_acc.at[dst_c], add=True)
        else:
            pltpu.async_copy(rowb, out_acc.at[dst_c], semS, add=True)

    def use(s):
        src_c, dst_c, rowb, exb, sem, semS = s
        pltpu.make_async_copy(h_hbm.at[src_c], rowb, sem).wait()
        pltpu.make_async_copy(ex.at[pl.ds(0, CH)], exb, sem).wait()
        compute(s)

    if nch % 2 == 1 and nch > 2:
        fire_s(0, set0, True)
        fire_s(1, set1, True)
        use(set0)
        fire_s(2, set0, False)
        use(set1)

        def pair(i, carry):
            fire_s(2 * i + 1, set1, False)
            use(set0)
            fire_s(2 * i + 2, set0, False)
            use(set1)
            return carry

        lax.fori_loop(1, (nch - 1) // 2, pair, 0)
        use(set0)
        drain_sc(set0)
        drain_sc(set1)
    else:
        def chunk(ch, carry):
            fire_s(ch, set0, True)
            src_c, dst_c, rowb, exb, sem, semS = set0
            pltpu.make_async_copy(h_hbm.at[src_c], rowb, sem).wait()
            pltpu.make_async_copy(ex.at[pl.ds(0, CH)], exb, sem).wait()
            compute(set0, sync_scatter=True)
            return carry

        lax.fori_loop(0, nch, chunk, 0)
    plsc.subcore_barrier()
    pltpu.sync_copy(out_acc.at[pl.ds(sid * rpt, rpt)],
                    out_parts.at[pl.ds(cid * npad + sid * rpt, rpt)])


def kernel(x, edge_index, W1, a_s1, a_d1, b1, W2, a_s2, a_d2, b2):
    n, d_in = x.shape
    d1 = W1.shape[1]
    d2 = W2.shape[1]
    npad = ((n + 1 + NS * 8 - 1) // (NS * 8)) * (NS * 8)
    rpt = npad // NS

    ei = edge_index.astype(jnp.int32)
    loops = jnp.arange(n, dtype=jnp.int32)
    src = jnp.concatenate([ei[0], loops])
    dst = jnp.concatenate([ei[1], loops])
    et = src.shape[0]
    nch = -(-et // (NW * CH))
    etp = NW * CH * nch
    src = jnp.concatenate([src, jnp.full((etp - et,), n, jnp.int32)])
    dst = jnp.concatenate([dst, jnp.full((etp - et,), n, jnp.int32)])
    srcp = src.reshape(NW, nch, CH)
    dstp = dst.reshape(NW, nch, CH)
    xp = jnp.pad(x, ((0, npad - n), (0, 0)))
    z128 = jnp.zeros((npad, d2), jnp.float32)
    b1r = b1.reshape(1, d1)
    b2r = b2.reshape(1, d2)
    as2v = a_s2.reshape(d2, 1)
    ad2v = a_d2.reshape(d2, 1)

    h1p, es1, ed1, g1 = pl.pallas_call(
        functools.partial(_tc0_body, n=n, npad=npad),
        out_shape=[jax.ShapeDtypeStruct((npad, 128), jnp.float32),
                   jax.ShapeDtypeStruct((npad, H1), jnp.float32),
                   jax.ShapeDtypeStruct((npad, H1), jnp.float32),
                   jax.ShapeDtypeStruct((1, 16), jnp.float32)],
    )(xp, W1, a_s1.reshape(1, d1), a_d1.reshape(1, d1))

    tab1 = jnp.concatenate(
        [jnp.concatenate([es1[:, 0:4], ed1[:, 0:4]], axis=1).reshape(-1),
         jnp.concatenate([es1[:, 4:8], ed1[:, 4:8]], axis=1).reshape(-1)])

    mesh = plsc.VectorSubcoreMesh(core_axis_name="c", subcore_axis_name="s")

    ex_lo, ex_hi = pl.kernel(
        functools.partial(_sc1_body, nch=nch, npad=npad),
        out_type=[jax.ShapeDtypeStruct((NW * nch * 4 * CH,), jnp.float32),
                  jax.ShapeDtypeStruct((NW * nch * 4 * CH,), jnp.float32)],
        mesh=mesh,
        compiler_params=pltpu.CompilerParams(needs_layout_passes=False),
        scratch_types=[
            pltpu.VMEM((nch, CH), jnp.int32),
            pltpu.VMEM((nch, CH), jnp.int32),
            pltpu.VMEM((npad * 8,), jnp.float32),
            pltpu.VMEM((4 * CH,), jnp.float32),
            pltpu.VMEM((16,), jnp.float32),
        ],
    )(srcp, dstp, tab1, g1.reshape(16))

    (out1_parts,) = pl.kernel(
        functools.partial(_sc2_body, nch=nch, rpt=rpt, npad=npad),
        out_type=[jax.ShapeDtypeStruct((2 * npad, 128), jnp.float32)],
        mesh=mesh,
        compiler_params=pltpu.CompilerParams(needs_layout_passes=False),
        scratch_types=[
            pltpu.VMEM((CH,), jnp.int32),
            pltpu.VMEM((CH,), jnp.int32),
            pltpu.VMEM((CH,), jnp.int32),
            pltpu.VMEM((CH,), jnp.int32),
            pltpu.VMEM((CH, 128), jnp.float32),
            pltpu.VMEM((CH, 128), jnp.float32),
            pltpu.VMEM((4 * CH,), jnp.float32),
            pltpu.VMEM((4 * CH,), jnp.float32),
            pltpu.VMEM((4 * CH,), jnp.float32),
            pltpu.VMEM((4 * CH,), jnp.float32),
            pltpu.VMEM_SHARED((npad, 128), jnp.float32),
            pltpu.SemaphoreType.DMA,
            pltpu.SemaphoreType.DMA,
            pltpu.SemaphoreType.DMA,
            pltpu.SemaphoreType.DMA,
        ],
    )(srcp.reshape(NW * nch, CH), dstp.reshape(NW * nch, CH),
      h1p, ex_lo, ex_hi, z128)

    h2, es2, ed2, g2 = pl.pallas_call(
        functools.partial(_merge1_body, n=n, npad=npad),
        out_shape=[jax.ShapeDtypeStruct((npad, d2), jnp.float32),
                   jax.ShapeDtypeStruct((npad, 1), jnp.float32),
                   jax.ShapeDtypeStruct((npad, 1), jnp.float32),
                   jax.ShapeDtypeStruct((1, 16), jnp.float32)],
    )(out1_parts, b1r, W2, as2v, ad2v)

    tab2 = jnp.concatenate([es2, ed2], axis=1).reshape(-1)

    ex2, s2_parts = pl.kernel(
        functools.partial(_sc3_body, nch=nch, rpt=rpt, npad=npad),
        out_type=[jax.ShapeDtypeStruct((NW * nch * CH,), jnp.float32),
                  jax.ShapeDtypeStruct((2 * npad, 128), jnp.float32)],
        mesh=mesh,
        compiler_params=pltpu.CompilerParams(needs_layout_passes=False),
        scratch_types=[
            pltpu.VMEM((CH,), jnp.int32),
            pltpu.VMEM((CH,), jnp.int32),
            pltpu.VMEM((npad * 2,), jnp.float32),
            pltpu.VMEM((CH,), jnp.float32),
            pltpu.VMEM((CH, 128), jnp.float32),
            pltpu.VMEM((16,), jnp.float32),
            pltpu.VMEM_SHARED((npad, 128), jnp.float32),
        ],
    )(srcp.reshape(NW * nch, CH), dstp.reshape(NW * nch, CH),
      tab2, g2.reshape(16), z128)

    (out2_parts,) = pl.kernel(
        functools.partial(_sc4_body, nch=nch, rpt=rpt, npad=npad),
        out_type=[jax.ShapeDtypeStruct((2 * npad, d2), jnp.float32)],
        mesh=mesh,
        compiler_params=pltpu.CompilerParams(needs_layout_passes=False),
        scratch_types=[
            pltpu.VMEM((CH,), jnp.int32),
            pltpu.VMEM((CH,), jnp.int32),
            pltpu.VMEM((CH,), jnp.int32),
            pltpu.VMEM((CH,), jnp.int32),
            pltpu.VMEM((CH, d2), jnp.float32),
            pltpu.VMEM((CH, d2), jnp.float32),
            pltpu.VMEM((CH,), jnp.float32),
            pltpu.VMEM((CH,), jnp.float32),
            pltpu.VMEM_SHARED((npad, d2), jnp.float32),
            pltpu.SemaphoreType.DMA,
            pltpu.SemaphoreType.DMA,
            pltpu.SemaphoreType.DMA,
            pltpu.SemaphoreType.DMA,
        ],
    )(srcp.reshape(NW * nch, CH), dstp.reshape(NW * nch, CH), h2, ex2, z128)

    (out,) = pl.pallas_call(
        functools.partial(_final_body, npad=npad),
        out_shape=[jax.ShapeDtypeStruct((npad, d2), jnp.float32)],
    )(s2_parts, out2_parts, b2r)
    return out[:n]


# speedup vs baseline: 1.0760x; 1.0760x over previous
import functools

import jax
import jax.numpy as jnp
from jax import lax
from jax.experimental import pallas as pl
from jax.experimental.pallas import tpu as pltpu
from jax.experimental.pallas import tpu_sc as plsc

NC, NS, L = 2, 16, 16
NW = NC * NS
CH = 128
NEG = -1e30
H1, C1 = 8, 8


def _perm(v, idx):
    dn = lax.GatherDimensionNumbers(
        offset_dims=(), collapsed_slice_dims=(0,), start_index_map=(0,))
    return lax.gather(v, idx[:, None], dn, slice_sizes=(1,),
                      mode=lax.GatherScatterMode.PROMISE_IN_BOUNDS)


def _splat(x):
    return jnp.full((L,), x, jnp.int32)


def _tc0_body(x_ref, w_ref, asf_ref, adf_ref, h_ref, es_ref, ed_ref, g_ref,
              *, n, npad):
    x = x_ref[...]
    h = jnp.dot(x, w_ref[...], preferred_element_type=jnp.float32)
    h_ref[...] = jnp.concatenate(
        [h, jnp.zeros((npad, 128 - H1 * C1), jnp.float32)], axis=1)
    kk = lax.broadcasted_iota(jnp.int32, (H1 * C1, H1), 0)
    hh = lax.broadcasted_iota(jnp.int32, (H1 * C1, H1), 1)
    blk = (kk // C1) == hh
    A_s = jnp.where(blk, jnp.broadcast_to(asf_ref[...].reshape(H1 * C1, 1),
                                          (H1 * C1, H1)), 0.0)
    A_d = jnp.where(blk, jnp.broadcast_to(adf_ref[...].reshape(H1 * C1, 1),
                                          (H1 * C1, H1)), 0.0)
    es = jnp.dot(h, A_s, preferred_element_type=jnp.float32)
    ed = jnp.dot(h, A_d, preferred_element_type=jnp.float32)
    valid = lax.broadcasted_iota(jnp.int32, (npad, H1), 0) < n
    esm = jnp.where(valid, es, NEG)
    edm = jnp.where(valid, ed, NEG)
    g = jnp.maximum(0.0, jnp.max(esm, axis=0) + jnp.max(edm, axis=0))
    g_ref[...] = jnp.concatenate([g, g])[None, :]
    es_ref[...] = esm
    ed_ref[...] = edm


def _merge1_body(op_ref, b1_ref, w2_ref, as2_ref, ad2_ref,
                 h2_ref, es2_ref, ed2_ref, g2_ref, *, n, npad):
    P = op_ref[0:npad] + op_ref[npad:2 * npad]
    acc = P[:, 0:H1 * C1]
    kk = lax.broadcasted_iota(jnp.int32, (128, H1), 0)
    hh = lax.broadcasted_iota(jnp.int32, (128, H1), 1)
    sel = kk == (64 + 8 * hh)
    Ssel = jnp.where(sel, 1.0, 0.0)
    s8 = jnp.dot(P, Ssel, preferred_element_type=jnp.float32)
    r8 = 1.0 / (s8 + 1e-16)
    hh2 = lax.broadcasted_iota(jnp.int32, (H1, H1 * C1), 0)
    kk2 = lax.broadcasted_iota(jnp.int32, (H1, H1 * C1), 1)
    Em = jnp.where(hh2 == (kk2 // C1), 1.0, 0.0)
    R = jnp.dot(r8, Em, preferred_element_type=jnp.float32)
    h2in = jnp.maximum(acc * R + b1_ref[...], 0.0)
    h2 = jnp.dot(h2in, w2_ref[...], preferred_element_type=jnp.float32)
    h2_ref[...] = h2
    es2 = jnp.dot(h2, as2_ref[...], preferred_element_type=jnp.float32)
    ed2 = jnp.dot(h2, ad2_ref[...], preferred_element_type=jnp.float32)
    valid = lax.broadcasted_iota(jnp.int32, (npad, 1), 0) < n
    es2m = jnp.where(valid, es2, NEG)
    ed2m = jnp.where(valid, ed2, NEG)
    g2 = jnp.maximum(0.0, jnp.max(es2m) + jnp.max(ed2m))
    g2_ref[...] = jnp.zeros((1, 16), jnp.float32) + g2
    es2_ref[...] = es2m
    ed2_ref[...] = ed2m


def _final_body(sp_ref, op_ref, b2_ref, o_ref, *, npad):
    s = jnp.sum(sp_ref[...], axis=0)[:, None]
    r = 1.0 / (s + 1e-16)
    acc = op_ref[0:npad] + op_ref[npad:2 * npad]
    out = acc * r + b2_ref[...]
    z = out - jnp.max(out, axis=1, keepdims=True)
    o_ref[...] = z - jnp.log(jnp.sum(jnp.exp(z), axis=1, keepdims=True))


def _sc1_body(srcp, dstp, tab_hbm, g, exlo_out, exhi_out,
              src_vm, dst_vm, tab_vm, exbuf, g_vm, *, nch, npad):
    cid = lax.axis_index("c")
    sid = lax.axis_index("s")
    wid = sid * NC + cid
    pltpu.sync_copy(srcp.at[wid], src_vm)
    pltpu.sync_copy(dstp.at[wid], dst_vm)
    pltpu.sync_copy(g, g_vm)
    gv = g_vm[...]
    wpn = 2 * 4
    for p, ex_out in ((0, exlo_out), (1, exhi_out)):
        pltpu.sync_copy(tab_hbm.at[pl.ds(p * npad * wpn, npad * wpn)], tab_vm)
        gj = [_perm(gv, _splat(4 * p + j)) for j in range(4)]

        def chunk(ch, carry):
            def group(gi, c2):
                src16 = src_vm[ch, pl.ds(gi * L, L)]
                dst16 = dst_vm[ch, pl.ds(gi * L, L)]
                for j in range(4):
                    es16 = plsc.load_gather(tab_vm, [src16 * wpn + j])
                    ed16 = plsc.load_gather(tab_vm, [dst16 * wpn + 4 + j])
                    t = es16 + ed16
                    t = jnp.maximum(t, 0.2 * t)
                    exbuf[pl.ds(j * CH + gi * L, L)] = jnp.exp(t - gj[j])
                return c2

            lax.fori_loop(0, CH // L, group, 0)
            pltpu.sync_copy(
                exbuf, ex_out.at[pl.ds((wid * nch + ch) * 4 * CH, 4 * CH)])
            return carry

        lax.fori_loop(0, nch, chunk, 0)


def _sc2_body(srcp2, dstp2, h_hbm, exlo, exhi, zD, out_parts,
              src_c0, dst_c0, src_c1, dst_c1, rowb0, rowb1,
              exl0, exh0, exl1, exh1, out_acc, sem0, sem1, semS0, semS1,
              *, nch, rpt, npad):
    cid = lax.axis_index("c")
    sid = lax.axis_index("s")
    wid = sid * NC + cid
    pltpu.sync_copy(zD.at[pl.ds(sid * rpt, rpt)],
                    out_acc.at[pl.ds(sid * rpt, rpt)])
    plsc.subcore_barrier()
    himask = lax.iota(jnp.int32, L) >= 8

    def fire(ch, src_c, dst_c, rowb, exl, exh, sem):
        pltpu.sync_copy(srcp2.at[wid * nch + ch], src_c)
        pltpu.sync_copy(dstp2.at[wid * nch + ch], dst_c)
        pltpu.async_copy(h_hbm.at[src_c], rowb, sem)
        pltpu.async_copy(exlo.at[pl.ds((wid * nch + ch) * 4 * CH, 4 * CH)],
                         exl, sem)
        pltpu.async_copy(exhi.at[pl.ds((wid * nch + ch) * 4 * CH, 4 * CH)],
                         exh, sem)

    def drain(src_c, rowb, exl, exh, sem):
        pltpu.make_async_copy(h_hbm.at[src_c], rowb, sem).wait()
        pltpu.make_async_copy(exlo.at[pl.ds(0, 4 * CH)], exl, sem).wait()
        pltpu.make_async_copy(exhi.at[pl.ds(0, 4 * CH)], exh, sem).wait()

    def compute(rowb, exl, exh, dst_c, semS):
        def group(gi, c2):
            eh = [exl[pl.ds(h * CH + gi * L, L)] for h in range(4)] + \
                 [exh[pl.ds(h * CH + gi * L, L)] for h in range(4)]
            for k in range(L):
                e = gi * L + k
                for j in range(4):
                    a = jnp.where(himask,
                                  _perm(eh[2 * j + 1], _splat(k)),
                                  _perm(eh[2 * j], _splat(k)))
                    rowb[e, pl.ds(j * L, L)] = rowb[e, pl.ds(j * L, L)] * a
                    rowb[e, pl.ds(64 + j * L, L)] = a
            return c2

        lax.fori_loop(0, CH // L, group, 0)
        pltpu.async_copy(rowb, out_acc.at[dst_c], semS, add=True)

    set0 = (src_c0, dst_c0, rowb0, exl0, exh0, sem0, semS0)
    set1 = (src_c1, dst_c1, rowb1, exl1, exh1, sem1, semS1)

    def drain_sc(s):
        src_c, dst_c, rowb, exl, exh, sem, semS = s
        pltpu.make_async_copy(rowb, out_acc.at[dst_c], semS).wait()

    def fire_s(ch, s, first):
        src_c, dst_c, rowb, exl, exh, sem, semS = s
        if not first:
            drain_sc(s)
        fire(ch, src_c, dst_c, rowb, exl, exh, sem)

    def use(s):
        src_c, dst_c, rowb, exl, exh, sem, semS = s
        drain(src_c, rowb, exl, exh, sem)
        compute(rowb, exl, exh, dst_c, semS)

    if nch % 2 == 1 and nch > 2:
        fire_s(0, set0, True)
        fire_s(1, set1, True)
        use(set0)
        fire_s(2, set0, False)
        use(set1)

        def pair(i, carry):
            fire_s(2 * i + 1, set1, False)
            use(set0)
            fire_s(2 * i + 2, set0, False)
            use(set1)
            return carry

        lax.fori_loop(1, (nch - 1) // 2, pair, 0)
        use(set0)
        drain_sc(set0)
        drain_sc(set1)
    else:
        def chunk(ch, carry):
            fire(ch, src_c0, dst_c0, rowb0, exl0, exh0, sem0)
            drain(src_c0, rowb0, exl0, exh0, sem0)

            def group(gi, c2):
                eh = [exl0[pl.ds(h * CH + gi * L, L)] for h in range(4)] + \
                     [exh0[pl.ds(h * CH + gi * L, L)] for h in range(4)]
                for k in range(L):
                    e = gi * L + k
                    for j in range(4):
                        a = jnp.where(himask,
                                      _perm(eh[2 * j + 1], _splat(k)),
                                      _perm(eh[2 * j], _splat(k)))
                        rowb0[e, pl.ds(j * L, L)] = \
                            rowb0[e, pl.ds(j * L, L)] * a
                        rowb0[e, pl.ds(64 + j * L, L)] = a
                return c2

            lax.fori_loop(0, CH // L, group, 0)
            pltpu.sync_copy(rowb0, out_acc.at[dst_c0], add=True)
            return carry

        lax.fori_loop(0, nch, chunk, 0)
    plsc.subcore_barrier()
    pltpu.sync_copy(out_acc.at[pl.ds(sid * rpt, rpt)],
                    out_parts.at[pl.ds(cid * npad + sid * rpt, rpt)])


def _sc3_body(srcp2, dstp2, tab_hbm, g, ex_out, s_parts,
              src_c, dst_c, tab_vm, exbuf, s2tab, g_vm,
              *, nch, rpt, npad):
    cid = lax.axis_index("c")
    sid = lax.axis_index("s")
    wid = sid * NC + cid
    pltpu.sync_copy(tab_hbm, tab_vm)
    pltpu.sync_copy(g, g_vm)

    def zr(i, c2):
        s2tab[pl.ds(i * L, L)] = jnp.zeros((L,), jnp.float32)
        return c2

    lax.fori_loop(0, npad // L, zr, 0)
    gv = g_vm[...]

    def chunk(ch, carry):
        pltpu.sync_copy(srcp2.at[wid * nch + ch], src_c)
        pltpu.sync_copy(dstp2.at[wid * nch + ch], dst_c)

        def group(gi, c2):
            src16 = src_c[pl.ds(gi * L, L)]
            dst16 = dst_c[pl.ds(gi * L, L)]
            es16 = plsc.load_gather(tab_vm, [src16 * 2])
            ed16 = plsc.load_gather(tab_vm, [dst16 * 2 + 1])
            t = es16 + ed16
            t = jnp.maximum(t, 0.2 * t)
            ex16 = jnp.exp(t - gv)
            exbuf[pl.ds(gi * L, L)] = ex16
            plsc.addupdate_scatter(s2tab, [dst16], ex16)
            return c2

        lax.fori_loop(0, CH // L, group, 0)
        pltpu.sync_copy(exbuf, ex_out.at[pl.ds((wid * nch + ch) * CH, CH)])
        return carry

    lax.fori_loop(0, nch, chunk, 0)
    pltpu.sync_copy(s2tab, s_parts.at[pl.ds(wid * npad, npad)])


def _sc4_body(srcp2, dstp2, h_hbm, ex, zD, out_parts,
              src_c0, dst_c0, src_c1, dst_c1, rowb0, rowb1,
              exb0, exb1, out_acc, sem0, sem1, semS0, semS1,
              *, nch, rpt, npad):
    cid = lax.axis_index("c")
    sid = lax.axis_index("s")
    wid = sid * NC + cid
    pltpu.sync_copy(zD.at[pl.ds(sid * rpt, rpt)],
                    out_acc.at[pl.ds(sid * rpt, rpt)])
    plsc.subcore_barrier()

    set0 = (src_c0, dst_c0, rowb0, exb0, sem0, semS0)
    set1 = (src_c1, dst_c1, rowb1, exb1, sem1, semS1)

    def drain_sc(s):
        src_c, dst_c, rowb, exb, sem, semS = s
        pltpu.make_async_copy(rowb, out_acc.at[dst_c], semS).wait()

    def fire_s(ch, s, first):
        src_c, dst_c, rowb, exb, sem, semS = s
        if not first:
            drain_sc(s)
        pltpu.sync_copy(srcp2.at[wid * nch + ch], src_c)
        pltpu.sync_copy(dstp2.at[wid * nch + ch], dst_c)
        pltpu.async_copy(h_hbm.at[src_c], rowb, sem)
        pltpu.async_copy(ex.at[pl.ds((wid * nch + ch) * CH, CH)], exb, sem)

    def compute(s, sync_scatter=False):
        src_c, dst_c, rowb, exb, sem, semS = s

        def group(gi, c2):
            exg = exb[pl.ds(gi * L, L)]
            for k in range(L):
                e = gi * L + k
                a = _perm(exg, _splat(k))
                for j in range(8):
                    rowb[e, pl.ds(j * L, L)] = rowb[e, pl.ds(j * L, L)] * a
            return c2

        lax.fori_loop(0, CH // L, group, 0)
        if sync_scatter:
            pltpu.sync_copy(rowb, out_acc.at[dst_c], add=True)
        else:
            pltpu.async_copy(rowb, out_acc.at[dst_c], semS, add=True)

    def use(s):
        src_c, dst_c, rowb, exb, sem, semS = s
        pltpu.make_async_copy(h_hbm.at[src_c], rowb, sem).wait()
        pltpu.make_async_copy(ex.at[pl.ds(0, CH)], exb, sem).wait()
        compute(s)

    if nch % 2 == 1 and nch > 2:
        fire_s(0, set0, True)
        fire_s(1, set1, True)
        use(set0)
        fire_s(2, set0, False)
        use(set1)

        def pair(i, carry):
            fire_s(2 * i + 1, set1, False)
            use(set0)
            fire_s(2 * i + 2, set0, False)
            use(set1)
            return carry

        lax.fori_loop(1, (nch - 1) // 2, pair, 0)
        use(set0)
        drain_sc(set0)
        drain_sc(set1)
    else:
        def chunk(ch, carry):
            fire_s(ch, set0, True)
            src_c, dst_c, rowb, exb, sem, semS = set0
            pltpu.make_async_copy(h_hbm.at[src_c], rowb, sem).wait()
            pltpu.make_async_copy(ex.at[pl.ds(0, CH)], exb, sem).wait()
            compute(set0, sync_scatter=True)
            return carry

        lax.fori_loop(0, nch, chunk, 0)
    plsc.subcore_barrier()
    pltpu.sync_copy(out_acc.at[pl.ds(sid * rpt, rpt)],
                    out_parts.at[pl.ds(cid * npad + sid * rpt, rpt)])


def kernel(x, edge_index, W1, a_s1, a_d1, b1, W2, a_s2, a_d2, b2):
    n, d_in = x.shape
    d1 = W1.shape[1]
    d2 = W2.shape[1]
    npad = ((n + 1 + NS * 8 - 1) // (NS * 8)) * (NS * 8)
    rpt = npad // NS

    ei = edge_index.astype(jnp.int32)
    loops = jnp.arange(n, dtype=jnp.int32)
    src = jnp.concatenate([ei[0], loops])
    dst = jnp.concatenate([ei[1], loops])
    et = src.shape[0]
    nch = -(-et // (NW * CH))
    etp = NW * CH * nch
    src = jnp.concatenate([src, jnp.full((etp - et,), n, jnp.int32)])
    dst = jnp.concatenate([dst, jnp.full((etp - et,), n, jnp.int32)])
    srcp = src.reshape(NW, nch, CH)
    dstp = dst.reshape(NW, nch, CH)
    xp = jnp.pad(x, ((0, npad - n), (0, 0)))
    z128 = jnp.zeros((npad, d2), jnp.float32)
    b1r = b1.reshape(1, d1)
    b2r = b2.reshape(1, d2)
    as2v = a_s2.reshape(d2, 1)
    ad2v = a_d2.reshape(d2, 1)

    h1p, es1, ed1, g1 = pl.pallas_call(
        functools.partial(_tc0_body, n=n, npad=npad),
        out_shape=[jax.ShapeDtypeStruct((npad, 128), jnp.float32),
                   jax.ShapeDtypeStruct((npad, H1), jnp.float32),
                   jax.ShapeDtypeStruct((npad, H1), jnp.float32),
                   jax.ShapeDtypeStruct((1, 16), jnp.float32)],
    )(xp, W1, a_s1.reshape(1, d1), a_d1.reshape(1, d1))

    tab1 = jnp.concatenate(
        [jnp.concatenate([es1[:, 0:4], ed1[:, 0:4]], axis=1).reshape(-1),
         jnp.concatenate([es1[:, 4:8], ed1[:, 4:8]], axis=1).reshape(-1)])

    mesh = plsc.VectorSubcoreMesh(core_axis_name="c", subcore_axis_name="s")

    ex_lo, ex_hi = pl.kernel(
        functools.partial(_sc1_body, nch=nch, npad=npad),
        out_type=[jax.ShapeDtypeStruct((NW * nch * 4 * CH,), jnp.float32),
                  jax.ShapeDtypeStruct((NW * nch * 4 * CH,), jnp.float32)],
        mesh=mesh,
        compiler_params=pltpu.CompilerParams(needs_layout_passes=False),
        scratch_types=[
            pltpu.VMEM((nch, CH), jnp.int32),
            pltpu.VMEM((nch, CH), jnp.int32),
            pltpu.VMEM((npad * 8,), jnp.float32),
            pltpu.VMEM((4 * CH,), jnp.float32),
            pltpu.VMEM((16,), jnp.float32),
        ],
    )(srcp, dstp, tab1, g1.reshape(16))

    (out1_parts,) = pl.kernel(
        functools.partial(_sc2_body, nch=nch, rpt=rpt, npad=npad),
        out_type=[jax.ShapeDtypeStruct((2 * npad, 128), jnp.float32)],
        mesh=mesh,
        compiler_params=pltpu.CompilerParams(needs_layout_passes=False),
        scratch_types=[
            pltpu.VMEM((CH,), jnp.int32),
            pltpu.VMEM((CH,), jnp.int32),
            pltpu.VMEM((CH,), jnp.int32),
            pltpu.VMEM((CH,), jnp.int32),
            pltpu.VMEM((CH, 128), jnp.float32),
            pltpu.VMEM((CH, 128), jnp.float32),
            pltpu.VMEM((4 * CH,), jnp.float32),
            pltpu.VMEM((4 * CH,), jnp.float32),
            pltpu.VMEM((4 * CH,), jnp.float32),
            pltpu.VMEM((4 * CH,), jnp.float32),
            pltpu.VMEM_SHARED((npad, 128), jnp.float32),
            pltpu.SemaphoreType.DMA,
            pltpu.SemaphoreType.DMA,
            pltpu.SemaphoreType.DMA,
            pltpu.SemaphoreType.DMA,
        ],
    )(srcp.reshape(NW * nch, CH), dstp.reshape(NW * nch, CH),
      h1p, ex_lo, ex_hi, z128)

    h2, es2, ed2, g2 = pl.pallas_call(
        functools.partial(_merge1_body, n=n, npad=npad),
        out_shape=[jax.ShapeDtypeStruct((npad, d2), jnp.float32),
                   jax.ShapeDtypeStruct((npad, 1), jnp.float32),
                   jax.ShapeDtypeStruct((npad, 1), jnp.float32),
                   jax.ShapeDtypeStruct((1, 16), jnp.float32)],
    )(out1_parts, b1r, W2, as2v, ad2v)

    tab2 = jnp.concatenate([es2, ed2], axis=1).reshape(-1)

    ex2, s2_parts = pl.kernel(
        functools.partial(_sc3_body, nch=nch, rpt=rpt, npad=npad),
        out_type=[jax.ShapeDtypeStruct((NW * nch * CH,), jnp.float32),
                  jax.ShapeDtypeStruct((NW * npad,), jnp.float32)],
        mesh=mesh,
        compiler_params=pltpu.CompilerParams(needs_layout_passes=False),
        scratch_types=[
            pltpu.VMEM((CH,), jnp.int32),
            pltpu.VMEM((CH,), jnp.int32),
            pltpu.VMEM((npad * 2,), jnp.float32),
            pltpu.VMEM((CH,), jnp.float32),
            pltpu.VMEM((npad,), jnp.float32),
            pltpu.VMEM((16,), jnp.float32),
        ],
    )(srcp.reshape(NW * nch, CH), dstp.reshape(NW * nch, CH),
      tab2, g2.reshape(16))

    (out2_parts,) = pl.kernel(
        functools.partial(_sc4_body, nch=nch, rpt=rpt, npad=npad),
        out_type=[jax.ShapeDtypeStruct((2 * npad, d2), jnp.float32)],
        mesh=mesh,
        compiler_params=pltpu.CompilerParams(needs_layout_passes=False),
        scratch_types=[
            pltpu.VMEM((CH,), jnp.int32),
            pltpu.VMEM((CH,), jnp.int32),
            pltpu.VMEM((CH,), jnp.int32),
            pltpu.VMEM((CH,), jnp.int32),
            pltpu.VMEM((CH, d2), jnp.float32),
            pltpu.VMEM((CH, d2), jnp.float32),
            pltpu.VMEM((CH,), jnp.float32),
            pltpu.VMEM((CH,), jnp.float32),
            pltpu.VMEM_SHARED((npad, d2), jnp.float32),
            pltpu.SemaphoreType.DMA,
            pltpu.SemaphoreType.DMA,
            pltpu.SemaphoreType.DMA,
            pltpu.SemaphoreType.DMA,
        ],
    )(srcp.reshape(NW * nch, CH), dstp.reshape(NW * nch, CH), h2, ex2, z128)

    (out,) = pl.pallas_call(
        functools.partial(_final_body, npad=npad),
        out_shape=[jax.ShapeDtypeStruct((npad, d2), jnp.float32)],
    )(s2_parts.reshape(NW, npad), out2_parts, b2r)
    return out[:n]
